# Initial kernel scaffold; baseline (speedup 1.0000x reference)
import functools

import jax
import jax.numpy as jnp
from jax import lax
from jax.experimental import pallas as pl
from jax.experimental.pallas import tpu as pltpu
from jax.experimental.pallas import tpu_sc as plsc

N = 10000
NP = 10016
E = 320000
NC = 2
NS = 16
NW = NC * NS
EPT = 10240
EP = NW * EPT
CH = 128
NCH = EPT // CH
RPT = NP // NS

_MESH = plsc.VectorSubcoreMesh(core_axis_name="c", subcore_axis_name="s")


def _make_prop(w):
    @functools.partial(
        pl.kernel,
        out_type=jax.ShapeDtypeStruct((NC, NP, w), jnp.float32),
        mesh=_MESH,
        scratch_types=[
            pltpu.VMEM((NCH, CH), jnp.int32),
            pltpu.VMEM((NCH, CH), jnp.int32),
            pltpu.VMEM((CH, w), jnp.float32),
            pltpu.VMEM((CH, w), jnp.float32),
            pltpu.VMEM_SHARED((NP, w), jnp.float32),
            pltpu.SemaphoreType.DMA,
            pltpu.SemaphoreType.DMA,
        ],
    )
    def prop(y_hbm, src_hbm, dst_hbm, zero_hbm, out_hbm,
             src_v, dst_v, buf0, buf1, acc, sem0, sem1):
        c = lax.axis_index("c")
        s = lax.axis_index("s")
        wid = c * NS + s
        r0 = s * RPT
        pltpu.sync_copy(zero_hbm.at[pl.ds(r0, RPT)], acc.at[pl.ds(r0, RPT)])
        pltpu.sync_copy(src_hbm.at[wid], src_v)
        pltpu.sync_copy(dst_hbm.at[wid], dst_v)
        plsc.subcore_barrier()

        pltpu.async_copy(y_hbm.at[src_v.at[0]], buf0, sem0)

        @pl.loop(0, NCH // 2)
        def _(j):
            e0 = 2 * j
            g1 = pltpu.async_copy(y_hbm.at[src_v.at[e0 + 1]], buf1, sem1)
            pltpu.make_async_copy(y_hbm.at[src_v.at[e0]], buf0, sem0).wait()
            pltpu.sync_copy(buf0, acc.at[dst_v.at[e0]], add=True)

            @pl.when(j < NCH // 2 - 1)
            def _():
                pltpu.async_copy(y_hbm.at[src_v.at[e0 + 2]], buf0, sem0)

            g1.wait()
            pltpu.sync_copy(buf1, acc.at[dst_v.at[e0 + 1]], add=True)

        plsc.subcore_barrier()
        pltpu.sync_copy(acc.at[pl.ds(r0, RPT)], out_hbm.at[c, pl.ds(r0, RPT)])

    return prop


_prop64 = _make_prop(64)
_prop1 = _make_prop(1)


@functools.partial(
    pl.kernel,
    out_type=jax.ShapeDtypeStruct((NC, NP, 1), jnp.float32),
    mesh=_MESH,
    scratch_types=[
        pltpu.VMEM((NCH, CH), jnp.int32),
        pltpu.VMEM((CH, 1), jnp.float32),
        pltpu.VMEM_SHARED((NP, 1), jnp.float32),
    ],
)
def _hist(dst_hbm, ones_hbm, zero_hbm, out_hbm, dst_v, ones_v, acc):
    c = lax.axis_index("c")
    s = lax.axis_index("s")
    wid = c * NS + s
    r0 = s * RPT
    pltpu.sync_copy(zero_hbm.at[pl.ds(r0, RPT)], acc.at[pl.ds(r0, RPT)])
    pltpu.sync_copy(dst_hbm.at[wid], dst_v)
    pltpu.sync_copy(ones_hbm, ones_v)
    plsc.subcore_barrier()

    @pl.loop(0, NCH)
    def _(j):
        pltpu.sync_copy(ones_v, acc.at[dst_v.at[j]], add=True)

    plsc.subcore_barrier()
    pltpu.sync_copy(acc.at[pl.ds(r0, RPT)], out_hbm.at[c, pl.ds(r0, RPT)])


def _tc_first(x, hist, w1):
    def body(x_ref, h_ref, w_ref, y_ref, d_ref):
        deg = h_ref[0] + h_ref[1] + 1.0
        dinv = lax.rsqrt(deg)
        xw = jnp.dot(x_ref[...], w_ref[...], preferred_element_type=jnp.float32)
        y_ref[...] = xw * dinv
        d_ref[...] = dinv

    return pl.pallas_call(
        body,
        out_shape=(
            jax.ShapeDtypeStruct((NP, 64), jnp.float32),
            jax.ShapeDtypeStruct((NP, 1), jnp.float32),
        ),
    )(x, hist, w1)


def _tc_mid(agg, y, dinv, b, w):
    wout = w.shape[1]

    def body(a_ref, y_ref, d_ref, b_ref, w_ref, o_ref):
        pre = d_ref[...] * (a_ref[0] + a_ref[1] + y_ref[...]) + b_ref[...]
        h = jnp.maximum(pre, 0.0)
        o_ref[...] = jnp.dot(h, w_ref[...],
                             preferred_element_type=jnp.float32) * d_ref[...]

    return pl.pallas_call(
        body,
        out_shape=jax.ShapeDtypeStruct((NP, wout), jnp.float32),
    )(agg, y, dinv, b, w)


def _tc_last(agg, y, dinv, b):
    def body(a_ref, y_ref, d_ref, b_ref, o_ref):
        o_ref[...] = d_ref[...] * (a_ref[0] + a_ref[1] + y_ref[...]) + b_ref[...]

    return pl.pallas_call(
        body,
        out_shape=jax.ShapeDtypeStruct((NP, 1), jnp.float32),
    )(agg, y, dinv, b)


def kernel(x, edge_index, edge_attr, W1, b1, W2, b2, W3, b3):
    del edge_attr
    pad = jnp.full((EP - E,), N, dtype=jnp.int32)
    src = jnp.concatenate([edge_index[0], pad]).reshape(NW, NCH, CH)
    dst = jnp.concatenate([edge_index[1], pad]).reshape(NW, NCH, CH)
    x_pad = jnp.pad(x, ((0, NP - N), (0, 0)))

    zeros64 = jnp.zeros((NP, 64), jnp.float32)
    zeros1 = jnp.zeros((NP, 1), jnp.float32)
    ones = jnp.ones((CH, 1), jnp.float32)

    hist = _hist(dst, ones, zeros1)
    y1, dinv = _tc_first(x_pad, hist, W1)
    agg1 = _prop64(y1, src, dst, zeros64)
    y2 = _tc_mid(agg1, y1, dinv, b1.reshape(1, 64), W2)
    agg2 = _prop64(y2, src, dst, zeros64)
    y3 = _tc_mid(agg2, y2, dinv, b2.reshape(1, 64), W3)
    agg3 = _prop1(y3, src, dst, zeros1)
    out = _tc_last(agg3, y3, dinv, b3.reshape(1, 1))
    return out[:N]


# speedup vs baseline: 18.4607x; 18.4607x over previous
import functools

import jax
import jax.numpy as jnp
from jax import lax
from jax.experimental import pallas as pl
from jax.experimental.pallas import tpu as pltpu
from jax.experimental.pallas import tpu_sc as plsc

N = 10000
NP = 10112
E = 320000
NC = 2
NS = 16
NW = NC * NS
EPT = 10240
EP = NW * EPT
CH = 128
NCH = EPT // CH
RPT = NP // NS

_MESH = plsc.VectorSubcoreMesh(core_axis_name="c", subcore_axis_name="s")


def _make_prop(w):
    @functools.partial(
        pl.kernel,
        out_type=jax.ShapeDtypeStruct((NC, NP, w), jnp.float32),
        mesh=_MESH,
        scratch_types=[
            pltpu.VMEM((NCH, CH), jnp.int32),
            pltpu.VMEM((NCH, CH), jnp.int32),
            pltpu.VMEM((CH, w), jnp.float32),
            pltpu.VMEM((CH, w), jnp.float32),
            pltpu.VMEM_SHARED((NP, w), jnp.float32),
            pltpu.SemaphoreType.DMA,
            pltpu.SemaphoreType.DMA,
        ],
        compiler_params=pltpu.CompilerParams(use_tc_tiling_on_sc=False),
    )
    def prop(y_hbm, src_hbm, dst_hbm, zero_hbm, out_hbm,
             src_v, dst_v, buf0, buf1, acc, sem0, sem1):
        c = lax.axis_index("c")
        s = lax.axis_index("s")
        wid = c * NS + s
        r0 = s * RPT
        pltpu.sync_copy(zero_hbm.at[pl.ds(r0, RPT)], acc.at[pl.ds(r0, RPT)])
        pltpu.sync_copy(src_hbm.at[wid], src_v)
        pltpu.sync_copy(dst_hbm.at[wid], dst_v)
        plsc.subcore_barrier()

        pltpu.async_copy(y_hbm.at[src_v.at[0]], buf0, sem0)

        @pl.loop(0, NCH // 2)
        def _(j):
            e0 = 2 * j
            g1 = pltpu.async_copy(y_hbm.at[src_v.at[e0 + 1]], buf1, sem1)
            pltpu.make_async_copy(y_hbm.at[src_v.at[e0]], buf0, sem0).wait()
            pltpu.sync_copy(buf0, acc.at[dst_v.at[e0]], add=True)

            @pl.when(j < NCH // 2 - 1)
            def _():
                pltpu.async_copy(y_hbm.at[src_v.at[e0 + 2]], buf0, sem0)

            g1.wait()
            pltpu.sync_copy(buf1, acc.at[dst_v.at[e0 + 1]], add=True)

        plsc.subcore_barrier()
        pltpu.sync_copy(acc.at[pl.ds(r0, RPT)], out_hbm.at[c, pl.ds(r0, RPT)])

    return prop


_prop64 = _make_prop(64)
_prop16 = _make_prop(16)


@functools.partial(
    pl.kernel,
    out_type=jax.ShapeDtypeStruct((NC, NP, 16), jnp.float32),
    mesh=_MESH,
    scratch_types=[
        pltpu.VMEM((NCH, CH), jnp.int32),
        pltpu.VMEM((CH, 16), jnp.float32),
        pltpu.VMEM_SHARED((NP, 16), jnp.float32),
    ],
    compiler_params=pltpu.CompilerParams(use_tc_tiling_on_sc=False),
)
def _hist(dst_hbm, ones_hbm, zero_hbm, out_hbm, dst_v, ones_v, acc):
    c = lax.axis_index("c")
    s = lax.axis_index("s")
    wid = c * NS + s
    r0 = s * RPT
    pltpu.sync_copy(zero_hbm.at[pl.ds(r0, RPT)], acc.at[pl.ds(r0, RPT)])
    pltpu.sync_copy(dst_hbm.at[wid], dst_v)
    pltpu.sync_copy(ones_hbm, ones_v)
    plsc.subcore_barrier()

    @pl.loop(0, NCH)
    def _(j):
        pltpu.sync_copy(ones_v, acc.at[dst_v.at[j]], add=True)

    plsc.subcore_barrier()
    pltpu.sync_copy(acc.at[pl.ds(r0, RPT)], out_hbm.at[c, pl.ds(r0, RPT)])


def _tc_first(x, hist, w1):
    def body(x_ref, h_ref, w_ref, y_ref, d_ref):
        deg = h_ref[0, :, 0:1] + h_ref[1, :, 0:1] + 1.0
        dinv = lax.rsqrt(deg)
        xw = jnp.dot(x_ref[...], w_ref[...], preferred_element_type=jnp.float32)
        y_ref[...] = xw * dinv
        d_ref[...] = dinv

    return pl.pallas_call(
        body,
        out_shape=(
            jax.ShapeDtypeStruct((NP, 64), jnp.float32),
            jax.ShapeDtypeStruct((NP, 1), jnp.float32),
        ),
    )(x, hist, w1)


def _tc_mid(agg, y, dinv, b, w):
    wout = w.shape[1]

    def body(a_ref, y_ref, d_ref, b_ref, w_ref, o_ref):
        pre = d_ref[...] * (a_ref[0] + a_ref[1] + y_ref[...]) + b_ref[...]
        h = jnp.maximum(pre, 0.0)
        o_ref[...] = jnp.dot(h, w_ref[...],
                             preferred_element_type=jnp.float32) * d_ref[...]

    return pl.pallas_call(
        body,
        out_shape=jax.ShapeDtypeStruct((NP, wout), jnp.float32),
    )(agg, y, dinv, b, w)


def _tc_last(agg, y, dinv, b):
    def body(a_ref, y_ref, d_ref, b_ref, o_ref):
        s = a_ref[0, :, 0:1] + a_ref[1, :, 0:1] + y_ref[:, 0:1]
        o_ref[...] = d_ref[...] * s + b_ref[...]

    return pl.pallas_call(
        body,
        out_shape=jax.ShapeDtypeStruct((NP, 1), jnp.float32),
    )(agg, y, dinv, b)


def kernel(x, edge_index, edge_attr, W1, b1, W2, b2, W3, b3):
    del edge_attr
    pad = jnp.full((EP - E,), N, dtype=jnp.int32)
    src = jnp.concatenate([edge_index[0], pad]).reshape(NW, NCH, CH)
    dst = jnp.concatenate([edge_index[1], pad]).reshape(NW, NCH, CH)
    x_pad = jnp.pad(x, ((0, NP - N), (0, 0)))

    zeros64 = jnp.zeros((NP, 64), jnp.float32)
    zeros16 = jnp.zeros((NP, 16), jnp.float32)
    ones = jnp.ones((CH, 16), jnp.float32)
    w3_pad = jnp.pad(W3, ((0, 0), (0, 16 - W3.shape[1])))

    hist = _hist(dst, ones, zeros16)
    y1, dinv = _tc_first(x_pad, hist, W1)
    agg1 = _prop64(y1, src, dst, zeros64)
    y2 = _tc_mid(agg1, y1, dinv, b1.reshape(1, 64), W2)
    agg2 = _prop64(y2, src, dst, zeros64)
    y3 = _tc_mid(agg2, y2, dinv, b2.reshape(1, 64), w3_pad)
    agg3 = _prop16(y3, src, dst, zeros16)
    out = _tc_last(agg3, y3, dinv, b3.reshape(1, 1))
    return out[:N]


# speedup vs baseline: 37.5948x; 2.0365x over previous
import functools

import jax
import jax.numpy as jnp
from jax import lax
from jax.experimental import pallas as pl
from jax.experimental.pallas import tpu as pltpu
from jax.experimental.pallas import tpu_sc as plsc

N = 10000
NP = 10112
E = 320000
NC = 2
NS = 16
NW = NC * NS
EPT = 10240
EP = NW * EPT
CH = 128
NCH = EPT // CH
RPT = NP // NS

_MESH = plsc.VectorSubcoreMesh(core_axis_name="c", subcore_axis_name="s")


def _make_prop(w):
    @functools.partial(
        pl.kernel,
        out_type=jax.ShapeDtypeStruct((NC, NP, w), jnp.float32),
        mesh=_MESH,
        scratch_types=[
            pltpu.VMEM((NCH, CH), jnp.int32),
            pltpu.VMEM((NCH, CH), jnp.int32),
            pltpu.VMEM((CH, w), jnp.float32),
            pltpu.VMEM((CH, w), jnp.float32),
            pltpu.VMEM_SHARED((NP, w), jnp.float32),
            pltpu.VMEM_SHARED((NP, w), jnp.float32),
            pltpu.SemaphoreType.DMA,
            pltpu.SemaphoreType.DMA,
        ],
        compiler_params=pltpu.CompilerParams(use_tc_tiling_on_sc=False),
    )
    def prop(y_hbm, src_hbm, dst_hbm, zero_hbm, out_hbm,
             src_v, dst_v, buf0, buf1, acc, ytab, sem0, sem1):
        c = lax.axis_index("c")
        s = lax.axis_index("s")
        wid = c * NS + s
        r0 = s * RPT
        pltpu.sync_copy(zero_hbm.at[pl.ds(r0, RPT)], acc.at[pl.ds(r0, RPT)])
        pltpu.sync_copy(y_hbm.at[pl.ds(r0, RPT)], ytab.at[pl.ds(r0, RPT)])
        pltpu.sync_copy(src_hbm.at[wid], src_v)
        pltpu.sync_copy(dst_hbm.at[wid], dst_v)
        plsc.subcore_barrier()

        pltpu.async_copy(ytab.at[src_v.at[0]], buf0, sem0)

        @pl.loop(0, NCH // 2)
        def _(j):
            e0 = 2 * j
            g1 = pltpu.async_copy(ytab.at[src_v.at[e0 + 1]], buf1, sem1)
            pltpu.make_async_copy(ytab.at[src_v.at[e0]], buf0, sem0).wait()
            pltpu.sync_copy(buf0, acc.at[dst_v.at[e0]], add=True)

            @pl.when(j < NCH // 2 - 1)
            def _():
                pltpu.async_copy(ytab.at[src_v.at[e0 + 2]], buf0, sem0)

            g1.wait()
            pltpu.sync_copy(buf1, acc.at[dst_v.at[e0 + 1]], add=True)

        plsc.subcore_barrier()
        pltpu.sync_copy(acc.at[pl.ds(r0, RPT)], out_hbm.at[c, pl.ds(r0, RPT)])

    return prop


_prop64 = _make_prop(64)
_prop16 = _make_prop(16)


@functools.partial(
    pl.kernel,
    out_type=jax.ShapeDtypeStruct((NC, NP, 16), jnp.float32),
    mesh=_MESH,
    scratch_types=[
        pltpu.VMEM((NCH, CH), jnp.int32),
        pltpu.VMEM((CH, 16), jnp.float32),
        pltpu.VMEM_SHARED((NP, 16), jnp.float32),
    ],
    compiler_params=pltpu.CompilerParams(use_tc_tiling_on_sc=False),
)
def _hist(dst_hbm, ones_hbm, zero_hbm, out_hbm, dst_v, ones_v, acc):
    c = lax.axis_index("c")
    s = lax.axis_index("s")
    wid = c * NS + s
    r0 = s * RPT
    pltpu.sync_copy(zero_hbm.at[pl.ds(r0, RPT)], acc.at[pl.ds(r0, RPT)])
    pltpu.sync_copy(dst_hbm.at[wid], dst_v)
    pltpu.sync_copy(ones_hbm, ones_v)
    plsc.subcore_barrier()

    @pl.loop(0, NCH)
    def _(j):
        pltpu.sync_copy(ones_v, acc.at[dst_v.at[j]], add=True)

    plsc.subcore_barrier()
    pltpu.sync_copy(acc.at[pl.ds(r0, RPT)], out_hbm.at[c, pl.ds(r0, RPT)])


def _tc_first(x, hist, w1):
    def body(x_ref, h_ref, w_ref, y_ref, d_ref):
        deg = h_ref[0, :, 0:1] + h_ref[1, :, 0:1] + 1.0
        dinv = lax.rsqrt(deg)
        xw = jnp.dot(x_ref[...], w_ref[...], preferred_element_type=jnp.float32)
        y_ref[...] = xw * dinv
        d_ref[...] = dinv

    return pl.pallas_call(
        body,
        out_shape=(
            jax.ShapeDtypeStruct((NP, 64), jnp.float32),
            jax.ShapeDtypeStruct((NP, 1), jnp.float32),
        ),
    )(x, hist, w1)


def _tc_mid(agg, y, dinv, b, w):
    wout = w.shape[1]

    def body(a_ref, y_ref, d_ref, b_ref, w_ref, o_ref):
        pre = d_ref[...] * (a_ref[0] + a_ref[1] + y_ref[...]) + b_ref[...]
        h = jnp.maximum(pre, 0.0)
        o_ref[...] = jnp.dot(h, w_ref[...],
                             preferred_element_type=jnp.float32) * d_ref[...]

    return pl.pallas_call(
        body,
        out_shape=jax.ShapeDtypeStruct((NP, wout), jnp.float32),
    )(agg, y, dinv, b, w)


def _tc_last(agg, y, dinv, b):
    def body(a_ref, y_ref, d_ref, b_ref, o_ref):
        s = a_ref[0, :, 0:1] + a_ref[1, :, 0:1] + y_ref[:, 0:1]
        o_ref[...] = d_ref[...] * s + b_ref[...]

    return pl.pallas_call(
        body,
        out_shape=jax.ShapeDtypeStruct((NP, 1), jnp.float32),
    )(agg, y, dinv, b)


def kernel(x, edge_index, edge_attr, W1, b1, W2, b2, W3, b3):
    del edge_attr
    pad = jnp.full((EP - E,), N, dtype=jnp.int32)
    src = jnp.concatenate([edge_index[0], pad]).reshape(NW, NCH, CH)
    dst = jnp.concatenate([edge_index[1], pad]).reshape(NW, NCH, CH)
    x_pad = jnp.pad(x, ((0, NP - N), (0, 0)))

    zeros64 = jnp.zeros((NP, 64), jnp.float32)
    zeros16 = jnp.zeros((NP, 16), jnp.float32)
    ones = jnp.ones((CH, 16), jnp.float32)
    w3_pad = jnp.pad(W3, ((0, 0), (0, 16 - W3.shape[1])))

    hist = _hist(dst, ones, zeros16)
    y1, dinv = _tc_first(x_pad, hist, W1)
    agg1 = _prop64(y1, src, dst, zeros64)
    y2 = _tc_mid(agg1, y1, dinv, b1.reshape(1, 64), W2)
    agg2 = _prop64(y2, src, dst, zeros64)
    y3 = _tc_mid(agg2, y2, dinv, b2.reshape(1, 64), w3_pad)
    agg3 = _prop16(y3, src, dst, zeros16)
    out = _tc_last(agg3, y3, dinv, b3.reshape(1, 1))
    return out[:N]


# speedup vs baseline: 42.1189x; 1.1203x over previous
import functools

import jax
import jax.numpy as jnp
from jax import lax
from jax.experimental import pallas as pl
from jax.experimental.pallas import tpu as pltpu
from jax.experimental.pallas import tpu_sc as plsc

N = 10000
E = 320000
NC = 2
NS = 16
NW = NC * NS
CH = 80
NCH = E // (NW * CH)
RPT = N // NS

_MESH = plsc.VectorSubcoreMesh(core_axis_name="c", subcore_axis_name="s")
_SC_PARAMS = pltpu.CompilerParams(use_tc_tiling_on_sc=False)


def _make_prop(w):
    @functools.partial(
        pl.kernel,
        out_type=jax.ShapeDtypeStruct((N, 128), jnp.float32),
        mesh=_MESH,
        scratch_types=[
            pltpu.VMEM((NCH, CH), jnp.int32),
            pltpu.VMEM((NCH, CH), jnp.int32),
            pltpu.VMEM((CH, w), jnp.float32),
            pltpu.VMEM((CH, w), jnp.float32),
            pltpu.VMEM_SHARED((N, w), jnp.float32),
            pltpu.VMEM_SHARED((N, w), jnp.float32),
            pltpu.SemaphoreType.DMA,
            pltpu.SemaphoreType.DMA,
        ],
        compiler_params=_SC_PARAMS,
    )
    def prop(y_hbm, src_hbm, dst_hbm, zero_hbm, out_hbm,
             src_v, dst_v, buf0, buf1, acc, ytab, sem0, sem1):
        c = lax.axis_index("c")
        s = lax.axis_index("s")
        wid = c * NS + s
        r0 = s * RPT
        pltpu.sync_copy(zero_hbm.at[pl.ds(r0, RPT)], acc.at[pl.ds(r0, RPT)])
        pltpu.sync_copy(y_hbm.at[pl.ds(r0, RPT)], ytab.at[pl.ds(r0, RPT)])
        pltpu.sync_copy(src_hbm.at[wid], src_v)
        pltpu.sync_copy(dst_hbm.at[wid], dst_v)
        plsc.subcore_barrier()

        pltpu.async_copy(ytab.at[src_v.at[0]], buf0, sem0)

        @pl.loop(0, NCH // 2)
        def _(j):
            e0 = 2 * j
            g1 = pltpu.async_copy(ytab.at[src_v.at[e0 + 1]], buf1, sem1)
            pltpu.make_async_copy(ytab.at[src_v.at[e0]], buf0, sem0).wait()
            pltpu.sync_copy(buf0, acc.at[dst_v.at[e0]], add=True)
            pltpu.async_copy(ytab.at[src_v.at[e0 + 2]], buf0, sem0)
            g1.wait()
            pltpu.sync_copy(buf1, acc.at[dst_v.at[e0 + 1]], add=True)

        pltpu.make_async_copy(ytab.at[src_v.at[NCH - 1]], buf0, sem0).wait()
        pltpu.sync_copy(buf0, acc.at[dst_v.at[NCH - 1]], add=True)

        plsc.subcore_barrier()
        pltpu.sync_copy(acc.at[pl.ds(r0, RPT)],
                        out_hbm.at[pl.ds(r0, RPT), pl.ds(64 * c, w)])

    return prop


_prop64 = _make_prop(64)
_prop16 = _make_prop(16)


@functools.partial(
    pl.kernel,
    out_type=jax.ShapeDtypeStruct((N, 128), jnp.float32),
    mesh=_MESH,
    scratch_types=[
        pltpu.VMEM((NCH, CH), jnp.int32),
        pltpu.VMEM((CH, 16), jnp.float32),
        pltpu.VMEM_SHARED((N, 16), jnp.float32),
    ],
    compiler_params=_SC_PARAMS,
)
def _hist(dst_hbm, ones_hbm, zero_hbm, out_hbm, dst_v, ones_v, acc):
    c = lax.axis_index("c")
    s = lax.axis_index("s")
    wid = c * NS + s
    r0 = s * RPT
    pltpu.sync_copy(zero_hbm.at[pl.ds(r0, RPT)], acc.at[pl.ds(r0, RPT)])
    pltpu.sync_copy(dst_hbm.at[wid], dst_v)
    pltpu.sync_copy(ones_hbm, ones_v)
    plsc.subcore_barrier()

    @pl.loop(0, NCH)
    def _(j):
        pltpu.sync_copy(ones_v, acc.at[dst_v.at[j]], add=True)

    plsc.subcore_barrier()
    pltpu.sync_copy(acc.at[pl.ds(r0, RPT)],
                    out_hbm.at[pl.ds(r0, RPT), pl.ds(64 * c, 16)])


def _tc_first(x, hist, w1):
    def body(x_ref, h_ref, w_ref, y_ref, d_ref):
        deg = h_ref[:, 0:1] + h_ref[:, 64:65] + 1.0
        dinv = lax.rsqrt(deg)
        xw = jnp.dot(x_ref[...], w_ref[...], preferred_element_type=jnp.float32)
        y_ref[...] = xw * dinv
        d_ref[...] = dinv

    return pl.pallas_call(
        body,
        out_shape=(
            jax.ShapeDtypeStruct((N, 64), jnp.float32),
            jax.ShapeDtypeStruct((N, 1), jnp.float32),
        ),
    )(x, hist, w1)


def _tc_mid(agg, y, dinv, b, w):
    wout = w.shape[1]

    def body(a_ref, y_ref, d_ref, b_ref, w_ref, o_ref):
        s = a_ref[:, 0:64] + a_ref[:, 64:128] + y_ref[...]
        h = jnp.maximum(d_ref[...] * s + b_ref[...], 0.0)
        o_ref[...] = jnp.dot(h, w_ref[...],
                             preferred_element_type=jnp.float32) * d_ref[...]

    return pl.pallas_call(
        body,
        out_shape=jax.ShapeDtypeStruct((N, wout), jnp.float32),
    )(agg, y, dinv, b, w)


def _tc_last(agg, y, dinv, b):
    def body(a_ref, y_ref, d_ref, b_ref, o_ref):
        s = a_ref[:, 0:1] + a_ref[:, 64:65] + y_ref[:, 0:1]
        o_ref[...] = d_ref[...] * s + b_ref[...]

    return pl.pallas_call(
        body,
        out_shape=jax.ShapeDtypeStruct((N, 1), jnp.float32),
    )(agg, y, dinv, b)


def kernel(x, edge_index, edge_attr, W1, b1, W2, b2, W3, b3):
    del edge_attr
    src = edge_index[0].reshape(NW, NCH, CH)
    dst = edge_index[1].reshape(NW, NCH, CH)

    zeros64 = jnp.zeros((N, 64), jnp.float32)
    zeros16 = jnp.zeros((N, 16), jnp.float32)
    ones = jnp.ones((CH, 16), jnp.float32)
    w3_pad = jnp.pad(W3, ((0, 0), (0, 16 - W3.shape[1])))

    hist = _hist(dst, ones, zeros16)
    y1, dinv = _tc_first(x, hist, W1)
    agg1 = _prop64(y1, src, dst, zeros64)
    y2 = _tc_mid(agg1, y1, dinv, b1.reshape(1, 64), W2)
    agg2 = _prop64(y2, src, dst, zeros64)
    y3 = _tc_mid(agg2, y2, dinv, b2.reshape(1, 64), w3_pad)
    agg3 = _prop16(y3, src, dst, zeros16)
    return _tc_last(agg3, y3, dinv, b3.reshape(1, 1))


# speedup vs baseline: 42.1366x; 1.0004x over previous
import functools

import jax
import jax.numpy as jnp
from jax import lax
from jax.experimental import pallas as pl
from jax.experimental.pallas import tpu as pltpu
from jax.experimental.pallas import tpu_sc as plsc

N = 10000
E = 320000
NC = 2
NS = 16
NW = NC * NS
CH = 80
NCH = E // (NW * CH)
EPW = E // NW
RPT = N // NS

_MESH = plsc.VectorSubcoreMesh(core_axis_name="c", subcore_axis_name="s")
_SC_PARAMS = pltpu.CompilerParams(use_tc_tiling_on_sc=False)


def _make_prop(w):
    @functools.partial(
        pl.kernel,
        out_type=jax.ShapeDtypeStruct((N, 128), jnp.float32),
        mesh=_MESH,
        scratch_types=[
            pltpu.VMEM((EPW,), jnp.int32),
            pltpu.VMEM((EPW,), jnp.int32),
            pltpu.VMEM((CH, w), jnp.float32),
            pltpu.VMEM((CH, w), jnp.float32),
            pltpu.VMEM_SHARED((N, w), jnp.float32),
            pltpu.VMEM_SHARED((N, w), jnp.float32),
            pltpu.SemaphoreType.DMA,
            pltpu.SemaphoreType.DMA,
        ],
        compiler_params=_SC_PARAMS,
    )
    def prop(y_hbm, src_hbm, dst_hbm, zero_hbm, out_hbm,
             src_v, dst_v, buf0, buf1, acc, ytab, sem0, sem1):
        c = lax.axis_index("c")
        s = lax.axis_index("s")
        wid = c * NS + s
        r0 = s * RPT
        pltpu.sync_copy(zero_hbm.at[pl.ds(r0, RPT)], acc.at[pl.ds(r0, RPT)])
        pltpu.sync_copy(y_hbm.at[pl.ds(r0, RPT)], ytab.at[pl.ds(r0, RPT)])
        pltpu.sync_copy(src_hbm.at[pl.ds(wid * EPW, EPW)], src_v)
        pltpu.sync_copy(dst_hbm.at[pl.ds(wid * EPW, EPW)], dst_v)
        plsc.subcore_barrier()

        pltpu.async_copy(ytab.at[src_v.at[pl.ds(0, CH)]], buf0, sem0)

        @pl.loop(0, NCH // 2)
        def _(j):
            e0 = 2 * j * CH
            g1 = pltpu.async_copy(
                ytab.at[src_v.at[pl.ds(e0 + CH, CH)]], buf1, sem1)
            pltpu.make_async_copy(
                ytab.at[src_v.at[pl.ds(e0, CH)]], buf0, sem0).wait()
            pltpu.sync_copy(buf0, acc.at[dst_v.at[pl.ds(e0, CH)]], add=True)
            pltpu.async_copy(
                ytab.at[src_v.at[pl.ds(e0 + 2 * CH, CH)]], buf0, sem0)
            g1.wait()
            pltpu.sync_copy(buf1, acc.at[dst_v.at[pl.ds(e0 + CH, CH)]],
                            add=True)

        pltpu.make_async_copy(
            ytab.at[src_v.at[pl.ds(EPW - CH, CH)]], buf0, sem0).wait()
        pltpu.sync_copy(buf0, acc.at[dst_v.at[pl.ds(EPW - CH, CH)]], add=True)

        plsc.subcore_barrier()
        pltpu.sync_copy(acc.at[pl.ds(r0, RPT)],
                        out_hbm.at[pl.ds(r0, RPT), pl.ds(64 * c, w)])

    return prop


_prop64 = _make_prop(64)
_prop16 = _make_prop(16)


@functools.partial(
    pl.kernel,
    out_type=jax.ShapeDtypeStruct((N, 128), jnp.float32),
    mesh=_MESH,
    scratch_types=[
        pltpu.VMEM((EPW,), jnp.int32),
        pltpu.VMEM((CH, 16), jnp.float32),
        pltpu.VMEM_SHARED((N, 16), jnp.float32),
    ],
    compiler_params=_SC_PARAMS,
)
def _hist(dst_hbm, ones_hbm, zero_hbm, out_hbm, dst_v, ones_v, acc):
    c = lax.axis_index("c")
    s = lax.axis_index("s")
    wid = c * NS + s
    r0 = s * RPT
    pltpu.sync_copy(zero_hbm.at[pl.ds(r0, RPT)], acc.at[pl.ds(r0, RPT)])
    pltpu.sync_copy(dst_hbm.at[pl.ds(wid * EPW, EPW)], dst_v)
    pltpu.sync_copy(ones_hbm, ones_v)
    plsc.subcore_barrier()

    @pl.loop(0, NCH)
    def _(j):
        pltpu.sync_copy(ones_v, acc.at[dst_v.at[pl.ds(j * CH, CH)]], add=True)

    plsc.subcore_barrier()
    pltpu.sync_copy(acc.at[pl.ds(r0, RPT)],
                    out_hbm.at[pl.ds(r0, RPT), pl.ds(64 * c, 16)])


def _tc_first(x, hist, w1):
    def body(x_ref, h_ref, w_ref, y_ref, d_ref):
        deg = h_ref[:, 0:1] + h_ref[:, 64:65] + 1.0
        dinv = lax.rsqrt(deg)
        xw = jnp.dot(x_ref[...], w_ref[...], preferred_element_type=jnp.float32)
        y_ref[...] = xw * dinv
        d_ref[...] = dinv

    return pl.pallas_call(
        body,
        out_shape=(
            jax.ShapeDtypeStruct((N, 64), jnp.float32),
            jax.ShapeDtypeStruct((N, 1), jnp.float32),
        ),
    )(x, hist, w1)


def _tc_mid(agg, y, dinv, b, w):
    wout = w.shape[1]

    def body(a_ref, y_ref, d_ref, b_ref, w_ref, o_ref):
        s = a_ref[:, 0:64] + a_ref[:, 64:128] + y_ref[...]
        h = jnp.maximum(d_ref[...] * s + b_ref[...], 0.0)
        o_ref[...] = jnp.dot(h, w_ref[...],
                             preferred_element_type=jnp.float32) * d_ref[...]

    return pl.pallas_call(
        body,
        out_shape=jax.ShapeDtypeStruct((N, wout), jnp.float32),
    )(agg, y, dinv, b, w)


def _tc_last(agg, y, dinv, b):
    def body(a_ref, y_ref, d_ref, b_ref, o_ref):
        s = a_ref[:, 0:1] + a_ref[:, 64:65] + y_ref[:, 0:1]
        o_ref[...] = d_ref[...] * s + b_ref[...]

    return pl.pallas_call(
        body,
        out_shape=jax.ShapeDtypeStruct((N, 1), jnp.float32),
    )(agg, y, dinv, b)


def kernel(x, edge_index, edge_attr, W1, b1, W2, b2, W3, b3):
    del edge_attr
    src = edge_index[0]
    dst = edge_index[1]

    zeros64 = jnp.zeros((N, 64), jnp.float32)
    zeros16 = jnp.zeros((N, 16), jnp.float32)
    ones = jnp.ones((CH, 16), jnp.float32)
    w3_pad = jnp.pad(W3, ((0, 0), (0, 16 - W3.shape[1])))

    hist = _hist(dst, ones, zeros16)
    y1, dinv = _tc_first(x, hist, W1)
    agg1 = _prop64(y1, src, dst, zeros64)
    y2 = _tc_mid(agg1, y1, dinv, b1.reshape(1, 64), W2)
    agg2 = _prop64(y2, src, dst, zeros64)
    y3 = _tc_mid(agg2, y2, dinv, b2.reshape(1, 64), w3_pad)
    agg3 = _prop16(y3, src, dst, zeros16)
    return _tc_last(agg3, y3, dinv, b3.reshape(1, 1))


# speedup vs baseline: 43.1742x; 1.0246x over previous
import functools

import jax
import jax.numpy as jnp
from jax import lax
from jax.experimental import pallas as pl
from jax.experimental.pallas import tpu as pltpu
from jax.experimental.pallas import tpu_sc as plsc

N = 10000
E = 320000
NC = 2
NS = 16
NW = NC * NS
CH = 80
NCH = E // (NW * CH)
EPW = E // NW
RPT = N // NS

_MESH = plsc.VectorSubcoreMesh(core_axis_name="c", subcore_axis_name="s")
_SC_PARAMS = pltpu.CompilerParams(use_tc_tiling_on_sc=False)


def _make_prop(w):
    @functools.partial(
        pl.kernel,
        out_type=jax.ShapeDtypeStruct((N, 128), jnp.float32),
        mesh=_MESH,
        scratch_types=[
            pltpu.VMEM((EPW,), jnp.int32),
            pltpu.VMEM((EPW,), jnp.int32),
            pltpu.VMEM((CH, w), jnp.float32),
            pltpu.VMEM((CH, w), jnp.float32),
            pltpu.VMEM_SHARED((N, w), jnp.float32),
            pltpu.VMEM_SHARED((N, w), jnp.float32),
            pltpu.SemaphoreType.DMA,
            pltpu.SemaphoreType.DMA,
        ],
        compiler_params=_SC_PARAMS,
    )
    def prop(y_hbm, src_hbm, dst_hbm, zero_hbm, out_hbm,
             src_v, dst_v, buf0, buf1, acc, ytab, sem0, sem1):
        c = lax.axis_index("c")
        s = lax.axis_index("s")
        wid = c * NS + s
        r0 = s * RPT
        pltpu.sync_copy(zero_hbm.at[pl.ds(r0, RPT)], acc.at[pl.ds(r0, RPT)])
        pltpu.sync_copy(y_hbm.at[pl.ds(r0, RPT), pl.ds(0, w)],
                        ytab.at[pl.ds(r0, RPT)])
        pltpu.sync_copy(src_hbm.at[pl.ds(wid * EPW, EPW)], src_v)
        pltpu.sync_copy(dst_hbm.at[pl.ds(wid * EPW, EPW)], dst_v)
        plsc.subcore_barrier()

        pltpu.async_copy(ytab.at[src_v.at[pl.ds(0, CH)]], buf0, sem0)

        @pl.loop(0, NCH // 2)
        def _(j):
            e0 = 2 * j * CH
            g1 = pltpu.async_copy(
                ytab.at[src_v.at[pl.ds(e0 + CH, CH)]], buf1, sem1)
            pltpu.make_async_copy(
                ytab.at[src_v.at[pl.ds(e0, CH)]], buf0, sem0).wait()
            pltpu.sync_copy(buf0, acc.at[dst_v.at[pl.ds(e0, CH)]], add=True)
            pltpu.async_copy(
                ytab.at[src_v.at[pl.ds(e0 + 2 * CH, CH)]], buf0, sem0)
            g1.wait()
            pltpu.sync_copy(buf1, acc.at[dst_v.at[pl.ds(e0 + CH, CH)]],
                            add=True)

        pltpu.make_async_copy(
            ytab.at[src_v.at[pl.ds(EPW - CH, CH)]], buf0, sem0).wait()
        pltpu.sync_copy(buf0, acc.at[dst_v.at[pl.ds(EPW - CH, CH)]], add=True)

        plsc.subcore_barrier()
        pltpu.sync_copy(acc.at[pl.ds(r0, RPT)],
                        out_hbm.at[pl.ds(r0, RPT), pl.ds(64 * c, w)])

    return prop


_prop64 = _make_prop(64)
_prop16 = _make_prop(16)


@functools.partial(
    pl.kernel,
    out_type=jax.ShapeDtypeStruct((N, 128), jnp.float32),
    mesh=_MESH,
    scratch_types=[
        pltpu.VMEM((EPW,), jnp.int32),
        pltpu.VMEM((CH, 16), jnp.float32),
        pltpu.VMEM_SHARED((N, 16), jnp.float32),
    ],
    compiler_params=_SC_PARAMS,
)
def _hist(dst_hbm, ones_hbm, zero_hbm, out_hbm, dst_v, ones_v, acc):
    c = lax.axis_index("c")
    s = lax.axis_index("s")
    wid = c * NS + s
    r0 = s * RPT
    pltpu.sync_copy(zero_hbm.at[pl.ds(r0, RPT)], acc.at[pl.ds(r0, RPT)])
    pltpu.sync_copy(dst_hbm.at[pl.ds(wid * EPW, EPW)], dst_v)
    pltpu.sync_copy(ones_hbm, ones_v)
    plsc.subcore_barrier()

    @pl.loop(0, NCH)
    def _(j):
        pltpu.sync_copy(ones_v, acc.at[dst_v.at[pl.ds(j * CH, CH)]], add=True)

    plsc.subcore_barrier()
    pltpu.sync_copy(acc.at[pl.ds(r0, RPT)],
                    out_hbm.at[pl.ds(r0, RPT), pl.ds(64 * c, 16)])


def _tc_matmul(x, w1):
    def body(x_ref, w_ref, o_ref):
        o_ref[...] = jnp.dot(x_ref[...], w_ref[...],
                             preferred_element_type=jnp.float32)

    return pl.pallas_call(
        body,
        out_shape=jax.ShapeDtypeStruct((N, 64), jnp.float32),
    )(x, w1)


def _tc_scale(hist, xw):
    def body(h_ref, x_ref, y_ref, d_ref):
        deg = h_ref[:, 0:1] + h_ref[:, 64:65] + 1.0
        dinv = lax.rsqrt(deg)
        y_ref[:, 0:64] = x_ref[...] * dinv
        y_ref[:, 64:128] = jnp.zeros((N, 64), jnp.float32)
        d_ref[...] = dinv

    return pl.pallas_call(
        body,
        out_shape=(
            jax.ShapeDtypeStruct((N, 128), jnp.float32),
            jax.ShapeDtypeStruct((N, 1), jnp.float32),
        ),
    )(hist, xw)


def _tc_mid(agg, y, dinv, b, w, wout):
    def body(a_ref, y_ref, d_ref, b_ref, w_ref, o_ref):
        s = a_ref[:, 0:64] + a_ref[:, 64:128] + y_ref[:, 0:64]
        h = jnp.maximum(d_ref[...] * s + b_ref[...], 0.0)
        yn = jnp.dot(h, w_ref[...],
                     preferred_element_type=jnp.float32) * d_ref[...]
        if wout == 64:
            o_ref[:, 0:64] = yn
            o_ref[:, 64:128] = jnp.zeros((N, 64), jnp.float32)
        else:
            o_ref[...] = yn

    return pl.pallas_call(
        body,
        out_shape=jax.ShapeDtypeStruct((N, 128 if wout == 64 else wout),
                                       jnp.float32),
    )(agg, y, dinv, b, w)


def _tc_last(agg, y, dinv, b):
    def body(a_ref, y_ref, d_ref, b_ref, o_ref):
        s = a_ref[:, 0:1] + a_ref[:, 64:65] + y_ref[:, 0:1]
        o_ref[...] = d_ref[...] * s + b_ref[...]

    return pl.pallas_call(
        body,
        out_shape=jax.ShapeDtypeStruct((N, 1), jnp.float32),
    )(agg, y, dinv, b)


def kernel(x, edge_index, edge_attr, W1, b1, W2, b2, W3, b3):
    del edge_attr
    src = edge_index[0]
    dst = edge_index[1]

    zeros64 = jnp.zeros((N, 64), jnp.float32)
    zeros16 = jnp.zeros((N, 16), jnp.float32)
    ones = jnp.ones((CH, 16), jnp.float32)
    w3_pad = jnp.pad(W3, ((0, 0), (0, 16 - W3.shape[1])))

    hist = _hist(dst, ones, zeros16)
    xw = _tc_matmul(x, W1)
    y1, dinv = _tc_scale(hist, xw)
    agg1 = _prop64(y1, src, dst, zeros64)
    y2 = _tc_mid(agg1, y1, dinv, b1.reshape(1, 64), W2, 64)
    agg2 = _prop64(y2, src, dst, zeros64)
    y3 = _tc_mid(agg2, y2, dinv, b2.reshape(1, 64), w3_pad, 16)
    agg3 = _prop16(y3, src, dst, zeros16)
    return _tc_last(agg3, y3, dinv, b3.reshape(1, 1))


# speedup vs baseline: 44.9579x; 1.0413x over previous
import functools

import jax
import jax.numpy as jnp
from jax import lax
from jax.experimental import pallas as pl
from jax.experimental.pallas import tpu as pltpu
from jax.experimental.pallas import tpu_sc as plsc

N = 10000
E = 320000
NC = 2
NS = 16
NW = NC * NS
CH = 80
NCH = E // (NW * CH)
EPW = E // NW
RPT = N // NS

_MESH = plsc.VectorSubcoreMesh(core_axis_name="c", subcore_axis_name="s")
_SC_PARAMS = pltpu.CompilerParams(use_tc_tiling_on_sc=False)


def _make_prop(w):
    @functools.partial(
        pl.kernel,
        out_type=jax.ShapeDtypeStruct((N, 128), jnp.float32),
        mesh=_MESH,
        scratch_types=[
            pltpu.VMEM((EPW,), jnp.int32),
            pltpu.VMEM((EPW,), jnp.int32),
            pltpu.VMEM((CH, w), jnp.float32),
            pltpu.VMEM((CH, w), jnp.float32),
            pltpu.VMEM_SHARED((N, w), jnp.float32),
            pltpu.VMEM_SHARED((N, w), jnp.float32),
            pltpu.SemaphoreType.DMA,
            pltpu.SemaphoreType.DMA,
        ],
        compiler_params=_SC_PARAMS,
    )
    def prop(y_hbm, ei_hbm, zero_hbm, out_hbm,
             src_v, dst_v, buf0, buf1, acc, ytab, sem0, sem1):
        c = lax.axis_index("c")
        s = lax.axis_index("s")
        wid = c * NS + s
        r0 = s * RPT
        pltpu.sync_copy(zero_hbm.at[pl.ds(r0, RPT)], acc.at[pl.ds(r0, RPT)])
        pltpu.sync_copy(y_hbm.at[pl.ds(r0, RPT), pl.ds(0, w)],
                        ytab.at[pl.ds(r0, RPT)])
        pltpu.sync_copy(ei_hbm.at[0, pl.ds(wid * EPW, EPW)], src_v)
        pltpu.sync_copy(ei_hbm.at[1, pl.ds(wid * EPW, EPW)], dst_v)
        plsc.subcore_barrier()

        pltpu.async_copy(ytab.at[src_v.at[pl.ds(0, CH)]], buf0, sem0)

        @pl.loop(0, NCH // 2)
        def _(j):
            e0 = 2 * j * CH
            g1 = pltpu.async_copy(
                ytab.at[src_v.at[pl.ds(e0 + CH, CH)]], buf1, sem1)
            pltpu.make_async_copy(
                ytab.at[src_v.at[pl.ds(e0, CH)]], buf0, sem0).wait()
            pltpu.sync_copy(buf0, acc.at[dst_v.at[pl.ds(e0, CH)]], add=True)
            pltpu.async_copy(
                ytab.at[src_v.at[pl.ds(e0 + 2 * CH, CH)]], buf0, sem0)
            g1.wait()
            pltpu.sync_copy(buf1, acc.at[dst_v.at[pl.ds(e0 + CH, CH)]],
                            add=True)

        pltpu.make_async_copy(
            ytab.at[src_v.at[pl.ds(EPW - CH, CH)]], buf0, sem0).wait()
        pltpu.sync_copy(buf0, acc.at[dst_v.at[pl.ds(EPW - CH, CH)]], add=True)

        plsc.subcore_barrier()
        pltpu.sync_copy(acc.at[pl.ds(r0, RPT)],
                        out_hbm.at[pl.ds(r0, RPT), pl.ds(64 * c, w)])

    return prop


_prop64 = _make_prop(64)
_prop16 = _make_prop(16)


@functools.partial(
    pl.kernel,
    out_type=jax.ShapeDtypeStruct((N, 128), jnp.float32),
    mesh=_MESH,
    scratch_types=[
        pltpu.VMEM((EPW,), jnp.int32),
        pltpu.VMEM((CH, 16), jnp.float32),
        pltpu.VMEM_SHARED((N, 16), jnp.float32),
    ],
    compiler_params=_SC_PARAMS,
)
def _hist(ei_hbm, ones_hbm, zero_hbm, out_hbm, dst_v, ones_v, acc):
    c = lax.axis_index("c")
    s = lax.axis_index("s")
    wid = c * NS + s
    r0 = s * RPT
    pltpu.sync_copy(zero_hbm.at[pl.ds(r0, RPT)], acc.at[pl.ds(r0, RPT)])
    pltpu.sync_copy(ei_hbm.at[1, pl.ds(wid * EPW, EPW)], dst_v)
    pltpu.sync_copy(ones_hbm, ones_v)
    plsc.subcore_barrier()

    @pl.loop(0, NCH)
    def _(j):
        pltpu.sync_copy(ones_v, acc.at[dst_v.at[pl.ds(j * CH, CH)]], add=True)

    plsc.subcore_barrier()
    pltpu.sync_copy(acc.at[pl.ds(r0, RPT)],
                    out_hbm.at[pl.ds(r0, RPT), pl.ds(64 * c, 16)])


def _tc_matmul(x, w1):
    def body(x_ref, w_ref, o_ref):
        o_ref[...] = jnp.dot(x_ref[...], w_ref[...],
                             preferred_element_type=jnp.float32)

    return pl.pallas_call(
        body,
        out_shape=jax.ShapeDtypeStruct((N, 64), jnp.float32),
    )(x, w1)


def _tc_scale(hist, xw):
    def body(h_ref, x_ref, y_ref, d_ref):
        deg = h_ref[:, 0:1] + h_ref[:, 64:65] + 1.0
        dinv = lax.rsqrt(deg)
        y_ref[:, 0:64] = x_ref[...] * dinv
        d_ref[...] = dinv

    return pl.pallas_call(
        body,
        out_shape=(
            jax.ShapeDtypeStruct((N, 128), jnp.float32),
            jax.ShapeDtypeStruct((N, 1), jnp.float32),
        ),
    )(hist, xw)


def _tc_mid(agg, y, dinv, b, w, wout):
    def body(a_ref, y_ref, d_ref, b_ref, w_ref, o_ref):
        s = a_ref[:, 0:64] + a_ref[:, 64:128] + y_ref[:, 0:64]
        h = jnp.maximum(d_ref[...] * s + b_ref[...], 0.0)
        yn = jnp.dot(h, w_ref[...],
                     preferred_element_type=jnp.float32) * d_ref[...]
        if wout == 64:
            o_ref[:, 0:64] = yn
        else:
            o_ref[...] = yn

    return pl.pallas_call(
        body,
        out_shape=jax.ShapeDtypeStruct((N, 128 if wout == 64 else wout),
                                       jnp.float32),
    )(agg, y, dinv, b, w)


def _tc_last(agg, y, dinv, b):
    def body(a_ref, y_ref, d_ref, b_ref, o_ref):
        s = a_ref[:, 0:1] + a_ref[:, 64:65] + y_ref[:, 0:1]
        o_ref[...] = d_ref[...] * s + b_ref[...]

    return pl.pallas_call(
        body,
        out_shape=jax.ShapeDtypeStruct((N, 1), jnp.float32),
    )(agg, y, dinv, b)


def kernel(x, edge_index, edge_attr, W1, b1, W2, b2, W3, b3):
    del edge_attr
    zeros64 = jnp.zeros((N, 64), jnp.float32)
    zeros16 = jnp.zeros((N, 16), jnp.float32)
    ones = jnp.ones((CH, 16), jnp.float32)
    w3_pad = jnp.pad(W3, ((0, 0), (0, 16 - W3.shape[1])))

    hist = _hist(edge_index, ones, zeros16)
    xw = _tc_matmul(x, W1)
    y1, dinv = _tc_scale(hist, xw)
    agg1 = _prop64(y1, edge_index, zeros64)
    y2 = _tc_mid(agg1, y1, dinv, b1.reshape(1, 64), W2, 64)
    agg2 = _prop64(y2, edge_index, zeros64)
    y3 = _tc_mid(agg2, y2, dinv, b2.reshape(1, 64), w3_pad, 16)
    agg3 = _prop16(y3, edge_index, zeros16)
    return _tc_last(agg3, y3, dinv, b3.reshape(1, 1))


# speedup vs baseline: 45.2547x; 1.0066x over previous
import functools

import jax
import jax.numpy as jnp
from jax import lax
from jax.experimental import pallas as pl
from jax.experimental.pallas import tpu as pltpu
from jax.experimental.pallas import tpu_sc as plsc

N = 10000
E = 320000
NC = 2
NS = 16
NW = NC * NS
CH = 80
NCH = E // (NW * CH)
EPW = E // NW
RPT = N // NS

_MESH = plsc.VectorSubcoreMesh(core_axis_name="c", subcore_axis_name="s")
_SC_PARAMS = pltpu.CompilerParams(use_tc_tiling_on_sc=False)


def _make_prop(w):
    @functools.partial(
        pl.kernel,
        out_type=jax.ShapeDtypeStruct((N, 128), jnp.float32),
        mesh=_MESH,
        scratch_types=[
            pltpu.VMEM((EPW,), jnp.int32),
            pltpu.VMEM((EPW,), jnp.int32),
            pltpu.VMEM((CH, w), jnp.float32),
            pltpu.VMEM((CH, w), jnp.float32),
            pltpu.VMEM_SHARED((N, w), jnp.float32),
            pltpu.VMEM_SHARED((N, w), jnp.float32),
            pltpu.SemaphoreType.DMA,
            pltpu.SemaphoreType.DMA,
        ],
        compiler_params=_SC_PARAMS,
    )
    def prop(y_hbm, ei_hbm, zero_hbm, out_hbm,
             src_v, dst_v, buf0, buf1, acc, ytab, sem0, sem1):
        c = lax.axis_index("c")
        s = lax.axis_index("s")
        wid = c * NS + s
        r0 = s * RPT
        pltpu.sync_copy(zero_hbm.at[pl.ds(r0, RPT)], acc.at[pl.ds(r0, RPT)])
        pltpu.sync_copy(y_hbm.at[pl.ds(r0, RPT), pl.ds(0, w)],
                        ytab.at[pl.ds(r0, RPT)])
        pltpu.sync_copy(ei_hbm.at[0, pl.ds(wid * EPW, EPW)], src_v)
        pltpu.sync_copy(ei_hbm.at[1, pl.ds(wid * EPW, EPW)], dst_v)
        plsc.subcore_barrier()

        pltpu.async_copy(ytab.at[src_v.at[pl.ds(0, CH)]], buf0, sem0)

        @pl.loop(0, NCH // 2)
        def _(j):
            e0 = 2 * j * CH
            g1 = pltpu.async_copy(
                ytab.at[src_v.at[pl.ds(e0 + CH, CH)]], buf1, sem1)
            pltpu.make_async_copy(
                ytab.at[src_v.at[pl.ds(e0, CH)]], buf0, sem0).wait()
            pltpu.sync_copy(buf0, acc.at[dst_v.at[pl.ds(e0, CH)]], add=True)
            pltpu.async_copy(
                ytab.at[src_v.at[pl.ds(e0 + 2 * CH, CH)]], buf0, sem0)
            g1.wait()
            pltpu.sync_copy(buf1, acc.at[dst_v.at[pl.ds(e0 + CH, CH)]],
                            add=True)

        pltpu.make_async_copy(
            ytab.at[src_v.at[pl.ds(EPW - CH, CH)]], buf0, sem0).wait()
        pltpu.sync_copy(buf0, acc.at[dst_v.at[pl.ds(EPW - CH, CH)]], add=True)

        plsc.subcore_barrier()
        pltpu.sync_copy(acc.at[pl.ds(r0, RPT)],
                        out_hbm.at[pl.ds(r0, RPT), pl.ds(64 * c, w)])

    return prop


_prop64 = _make_prop(64)
_prop16 = _make_prop(16)


@functools.partial(
    pl.kernel,
    out_type=jax.ShapeDtypeStruct((N, 128), jnp.float32),
    mesh=_MESH,
    scratch_types=[
        pltpu.VMEM((EPW,), jnp.int32),
        pltpu.VMEM((CH, 16), jnp.float32),
        pltpu.VMEM_SHARED((N, 16), jnp.float32),
    ],
    compiler_params=_SC_PARAMS,
)
def _hist(ei_hbm, ones_hbm, zero_hbm, out_hbm, dst_v, ones_v, acc):
    c = lax.axis_index("c")
    s = lax.axis_index("s")
    wid = c * NS + s
    r0 = s * RPT
    pltpu.sync_copy(zero_hbm.at[pl.ds(r0, RPT)], acc.at[pl.ds(r0, RPT)])
    pltpu.sync_copy(ei_hbm.at[1, pl.ds(wid * EPW, EPW)], dst_v)
    pltpu.sync_copy(ones_hbm, ones_v)
    plsc.subcore_barrier()

    @pl.loop(0, NCH)
    def _(j):
        pltpu.sync_copy(ones_v, acc.at[dst_v.at[pl.ds(j * CH, CH)]], add=True)

    plsc.subcore_barrier()
    pltpu.sync_copy(acc.at[pl.ds(r0, RPT)],
                    out_hbm.at[pl.ds(r0, RPT), pl.ds(64 * c, 16)])


def _tc_matmul(x, w1):
    def body(x_ref, w_ref, o_ref):
        o_ref[...] = jnp.dot(x_ref[...], w_ref[...],
                             preferred_element_type=jnp.float32)

    return pl.pallas_call(
        body,
        out_shape=jax.ShapeDtypeStruct((N, 64), jnp.float32),
    )(x, w1)


def _tc_scale(hist, xw):
    def body(h_ref, x_ref, y_ref, d_ref):
        deg = h_ref[:, 0:1] + h_ref[:, 64:65] + 1.0
        dinv = lax.rsqrt(deg)
        y_ref[:, 0:64] = x_ref[...] * dinv
        d_ref[...] = dinv

    return pl.pallas_call(
        body,
        out_shape=(
            jax.ShapeDtypeStruct((N, 128), jnp.float32),
            jax.ShapeDtypeStruct((N, 1), jnp.float32),
        ),
    )(hist, xw)


def _tc_mid(agg, y, dinv, b, w, wout):
    def body(a_ref, y_ref, d_ref, b_ref, w_ref, o_ref):
        s = a_ref[:, 0:64] + a_ref[:, 64:128] + y_ref[:, 0:64]
        h = jnp.maximum(d_ref[...] * s + b_ref[...], 0.0)
        yn = jnp.dot(h, w_ref[...],
                     preferred_element_type=jnp.float32) * d_ref[...]
        o_ref[:, 0:wout] = yn

    return pl.pallas_call(
        body,
        out_shape=jax.ShapeDtypeStruct((N, 128), jnp.float32),
    )(agg, y, dinv, b, w)


def _tc_last(agg, y, dinv, b):
    def body(a_ref, y_ref, d_ref, b_ref, o_ref):
        s = a_ref[:, 0:1] + a_ref[:, 64:65] + y_ref[:, 0:1]
        o_ref[...] = d_ref[...] * s + b_ref[...]

    return pl.pallas_call(
        body,
        out_shape=jax.ShapeDtypeStruct((N, 1), jnp.float32),
    )(agg, y, dinv, b)


def kernel(x, edge_index, edge_attr, W1, b1, W2, b2, W3, b3):
    del edge_attr
    zeros64 = jnp.zeros((N, 64), jnp.float32)
    zeros16 = jnp.zeros((N, 16), jnp.float32)
    ones = jnp.ones((CH, 16), jnp.float32)
    w3_pad = jnp.pad(W3, ((0, 0), (0, 16 - W3.shape[1])))

    hist = _hist(edge_index, ones, zeros16)
    xw = _tc_matmul(x, W1)
    y1, dinv = _tc_scale(hist, xw)
    agg1 = _prop64(y1, edge_index, zeros64)
    y2 = _tc_mid(agg1, y1, dinv, b1.reshape(1, 64), W2, 64)
    agg2 = _prop64(y2, edge_index, zeros64)
    y3 = _tc_mid(agg2, y2, dinv, b2.reshape(1, 64), w3_pad, 16)
    agg3 = _prop16(y3, edge_index, zeros16)
    return _tc_last(agg3, y3, dinv, b3.reshape(1, 1))


# speedup vs baseline: 45.2829x; 1.0006x over previous
import functools

import jax
import jax.numpy as jnp
from jax import lax
from jax.experimental import pallas as pl
from jax.experimental.pallas import tpu as pltpu
from jax.experimental.pallas import tpu_sc as plsc

N = 10000
E = 320000
NC = 2
NS = 16
NW = NC * NS
CH = 80
NCH = E // (NW * CH)
EPW = E // NW
RPT = N // NS

_MESH = plsc.VectorSubcoreMesh(core_axis_name="c", subcore_axis_name="s",
                               num_cores=NC, num_subcores=NS)
_SC_PARAMS = pltpu.CompilerParams(use_tc_tiling_on_sc=False)


def _make_prop(w):
    @functools.partial(
        pl.kernel,
        out_type=jax.ShapeDtypeStruct((N, 128), jnp.float32),
        mesh=_MESH,
        scratch_types=[
            pltpu.VMEM((EPW,), jnp.int32),
            pltpu.VMEM((EPW,), jnp.int32),
            pltpu.VMEM((CH, w), jnp.float32),
            pltpu.VMEM((CH, w), jnp.float32),
            pltpu.VMEM_SHARED((N, w), jnp.float32),
            pltpu.VMEM_SHARED((N, w), jnp.float32),
            pltpu.SemaphoreType.DMA,
            pltpu.SemaphoreType.DMA,
        ],
        compiler_params=_SC_PARAMS,
    )
    def prop(y_hbm, ei_hbm, zero_hbm, out_hbm,
             src_v, dst_v, buf0, buf1, acc, ytab, sem0, sem1):
        c = lax.axis_index("c")
        s = lax.axis_index("s")
        wid = c * NS + s
        r0 = s * RPT
        pltpu.sync_copy(zero_hbm.at[pl.ds(r0, RPT)], acc.at[pl.ds(r0, RPT)])
        pltpu.sync_copy(y_hbm.at[pl.ds(r0, RPT), pl.ds(0, w)],
                        ytab.at[pl.ds(r0, RPT)])
        pltpu.sync_copy(ei_hbm.at[0, pl.ds(wid * EPW, EPW)], src_v)
        pltpu.sync_copy(ei_hbm.at[1, pl.ds(wid * EPW, EPW)], dst_v)
        plsc.subcore_barrier()

        pltpu.async_copy(ytab.at[src_v.at[pl.ds(0, CH)]], buf0, sem0)

        @pl.loop(0, NCH // 2)
        def _(j):
            e0 = 2 * j * CH
            g1 = pltpu.async_copy(
                ytab.at[src_v.at[pl.ds(e0 + CH, CH)]], buf1, sem1)
            pltpu.make_async_copy(
                ytab.at[src_v.at[pl.ds(e0, CH)]], buf0, sem0).wait()
            pltpu.sync_copy(buf0, acc.at[dst_v.at[pl.ds(e0, CH)]], add=True)
            pltpu.async_copy(
                ytab.at[src_v.at[pl.ds(e0 + 2 * CH, CH)]], buf0, sem0)
            g1.wait()
            pltpu.sync_copy(buf1, acc.at[dst_v.at[pl.ds(e0 + CH, CH)]],
                            add=True)

        pltpu.make_async_copy(
            ytab.at[src_v.at[pl.ds(EPW - CH, CH)]], buf0, sem0).wait()
        pltpu.sync_copy(buf0, acc.at[dst_v.at[pl.ds(EPW - CH, CH)]], add=True)

        plsc.subcore_barrier()
        pltpu.sync_copy(acc.at[pl.ds(r0, RPT)],
                        out_hbm.at[pl.ds(r0, RPT), pl.ds(64 * c, w)])

    return prop


_prop64 = _make_prop(64)
_prop16 = _make_prop(16)


@functools.partial(
    pl.kernel,
    out_type=jax.ShapeDtypeStruct((N, 128), jnp.float32),
    mesh=_MESH,
    scratch_types=[
        pltpu.VMEM((EPW,), jnp.int32),
        pltpu.VMEM((CH, 16), jnp.float32),
        pltpu.VMEM_SHARED((N, 16), jnp.float32),
    ],
    compiler_params=_SC_PARAMS,
)
def _hist(ei_hbm, ones_hbm, zero_hbm, out_hbm, dst_v, ones_v, acc):
    c = lax.axis_index("c")
    s = lax.axis_index("s")
    wid = c * NS + s
    r0 = s * RPT
    pltpu.sync_copy(zero_hbm.at[pl.ds(r0, RPT)], acc.at[pl.ds(r0, RPT)])
    pltpu.sync_copy(ei_hbm.at[1, pl.ds(wid * EPW, EPW)], dst_v)
    pltpu.sync_copy(ones_hbm, ones_v)
    plsc.subcore_barrier()

    @pl.loop(0, NCH)
    def _(j):
        pltpu.sync_copy(ones_v, acc.at[dst_v.at[pl.ds(j * CH, CH)]], add=True)

    plsc.subcore_barrier()
    pltpu.sync_copy(acc.at[pl.ds(r0, RPT)],
                    out_hbm.at[pl.ds(r0, RPT), pl.ds(64 * c, 16)])


def _tc_matmul(x, w1):
    def body(x_ref, w_ref, o_ref):
        o_ref[...] = jnp.dot(x_ref[...], w_ref[...],
                             preferred_element_type=jnp.float32)

    return pl.pallas_call(
        body,
        out_shape=jax.ShapeDtypeStruct((N, 64), jnp.float32),
    )(x, w1)


def _tc_scale(hist, xw):
    def body(h_ref, x_ref, y_ref, d_ref):
        deg = h_ref[:, 0:1] + h_ref[:, 64:65] + 1.0
        dinv = lax.rsqrt(deg)
        y_ref[:, 0:64] = x_ref[...] * dinv
        d_ref[...] = dinv

    return pl.pallas_call(
        body,
        out_shape=(
            jax.ShapeDtypeStruct((N, 128), jnp.float32),
            jax.ShapeDtypeStruct((N, 1), jnp.float32),
        ),
    )(hist, xw)


def _tc_mid(agg, y, dinv, b, w, wout):
    def body(a_ref, y_ref, d_ref, b_ref, w_ref, o_ref):
        s = a_ref[:, 0:64] + a_ref[:, 64:128] + y_ref[:, 0:64]
        h = jnp.maximum(d_ref[...] * s + b_ref[...], 0.0)
        yn = jnp.dot(h, w_ref[...],
                     preferred_element_type=jnp.float32) * d_ref[...]
        o_ref[:, 0:wout] = yn

    return pl.pallas_call(
        body,
        out_shape=jax.ShapeDtypeStruct((N, 128), jnp.float32),
    )(agg, y, dinv, b, w)


def _tc_last(agg, y, dinv, b):
    def body(a_ref, y_ref, d_ref, b_ref, o_ref):
        s = a_ref[:, 0:1] + a_ref[:, 64:65] + y_ref[:, 0:1]
        o_ref[...] = d_ref[...] * s + b_ref[...]

    return pl.pallas_call(
        body,
        out_shape=jax.ShapeDtypeStruct((N, 1), jnp.float32),
    )(agg, y, dinv, b)


def kernel(x, edge_index, edge_attr, W1, b1, W2, b2, W3, b3):
    del edge_attr
    zeros64 = jnp.zeros((N, 64), jnp.float32)
    zeros16 = jnp.zeros((N, 16), jnp.float32)
    ones = jnp.ones((CH, 16), jnp.float32)
    w3_pad = jnp.pad(W3, ((0, 0), (0, 16 - W3.shape[1])))

    hist = _hist(edge_index, ones, zeros16)
    xw = _tc_matmul(x, W1)
    y1, dinv = _tc_scale(hist, xw)
    agg1 = _prop64(y1, edge_index, zeros64)
    y2 = _tc_mid(agg1, y1, dinv, b1.reshape(1, 64), W2, 64)
    agg2 = _prop64(y2, edge_index, zeros64)
    y3 = _tc_mid(agg2, y2, dinv, b2.reshape(1, 64), w3_pad, 16)
    agg3 = _prop16(y3, edge_index, zeros16)
    return _tc_last(agg3, y3, dinv, b3.reshape(1, 1))
